# baseline (speedup 1.0000x reference)
import functools

import jax
import jax.numpy as jnp
from jax import lax
from jax.experimental import pallas as pl
from jax.experimental.pallas import tpu as pltpu
from jax.experimental.pallas import tpu_sc as plsc

NC = 2
NS = 16
NW = NC * NS

CH = 128
K = 4


def _gather_sc(idx_t, embedding, b, t, d):
    n_total = b * t
    n_units = n_total // CH
    n_su = n_units // K
    su_per_w = n_su // NW
    bh_n = b // CH
    dh_n = d // 8
    tbl = d * CH * K

    mesh = plsc.VectorSubcoreMesh(core_axis_name="c", subcore_axis_name="s")

    @functools.partial(
        pl.kernel,
        mesh=mesh,
        out_type=jax.ShapeDtypeStruct((n_total * d,), jnp.float32),
        compiler_params=pltpu.CompilerParams(
            use_tc_tiling_on_sc=False, needs_layout_passes=False),
        scratch_types=[
            pltpu.VMEM((su_per_w * K * CH,), jnp.int32),
            pltpu.VMEM((K * CH, d), jnp.float32),
            pltpu.VMEM((K * CH, d), jnp.float32),
            pltpu.VMEM((tbl,), jnp.float32),
            pltpu.VMEM((tbl,), jnp.float32),
            pltpu.SemaphoreType.DMA,
            pltpu.SemaphoreType.DMA,
            pltpu.SemaphoreType.DMA,
            pltpu.SemaphoreType.DMA,
        ],
    )
    def k(idx_hbm, table_hbm, out_hbm, idx_v, rows0, rows1, tb0, tb1,
          g0, g1, w0, w1):
        wid = lax.axis_index("s") * NC + lax.axis_index("c")
        sbase = wid * su_per_w
        pltpu.sync_copy(
            idx_hbm.at[pl.ds(sbase * K * CH, su_per_w * K * CH)], idx_v)

        iota = lax.iota(jnp.int32, 16)
        skews = [lax.rem(iota + dd, 16) for dd in range(16)]
        svs = [[((iota + c0) // 8) * (K * CH * 8) + ((iota + c0) % 8) * CH
                + skews[dd] for dd in range(16)]
               for c0 in range(0, d, 16)]

        def gather(ls, buf, sem):
            return pltpu.make_async_copy(
                table_hbm.at[idx_v.at[pl.ds(ls * (K * CH), K * CH)]], buf, sem)

        def transpose(rows, tb):
            for kk in range(K):
                def tblk(ii, _, kk=kk):
                    bi0 = ii * 16
                    for j, c0 in enumerate(range(0, d, 16)):
                        for dd in range(16):
                            rv = skews[dd] + (kk * CH + bi0)
                            v = plsc.load_gather(rows, [rv, iota + c0])
                            plsc.store_scatter(
                                tb, [svs[j][dd] + (kk * CH * 8 + bi0)], v)
                    return 0
                lax.fori_loop(0, CH // 16, tblk, 0)

        def writes(ls, tb, sem):
            u = (sbase + ls) * K
            row = (u // bh_n) * (dh_n * bh_n) + lax.rem(u, bh_n)
            return [pltpu.make_async_copy(
                        tb.at[pl.ds(ch * (K * CH * 8), K * CH * 8)],
                        out_hbm.at[pl.ds((row + ch * bh_n) * (8 * CH),
                                         K * CH * 8)],
                        sem)
                    for ch in range(dh_n)]

        gather(0, rows0, g0).start()
        gather(1, rows1, g1).start()

        def su(i, ls, rows, tb, gsem, wsem):
            gather(ls, rows, gsem).wait()
            transpose(rows, tb)
            gather(lax.rem(ls + 2, su_per_w), rows, gsem).start()
            for cp in writes(ls, tb, wsem):
                cp.start()

        def body(i, _):
            ls = 2 * i
            su(i, ls, rows0, tb0, g0, w0)
            su(i, ls + 1, rows1, tb1, g1, w1)
            for cp in writes(ls, tb0, w0) + writes(ls + 1, tb1, w1):
                cp.wait()
            return 0

        lax.fori_loop(0, su_per_w // 2, body, 0)

        gather(0, rows0, g0).wait()
        gather(1, rows1, g1).wait()

    return k(idx_t, embedding)


def kernel(state, embedding):
    b, t = state.shape
    _, d = embedding.shape
    idx_t = state.T.reshape(b * t)
    out1 = _gather_sc(idx_t, embedding, b, t, d)
    out5 = out1.reshape(t, d // 8, b // CH, 8, CH)
    return out5.transpose(2, 4, 0, 1, 3).reshape(b, t, d)


# speedup vs baseline: 1.0104x; 1.0104x over previous
import functools

import jax
import jax.numpy as jnp
from jax import lax
from jax.experimental import pallas as pl
from jax.experimental.pallas import tpu as pltpu
from jax.experimental.pallas import tpu_sc as plsc

NC = 2
NS = 16
NW = NC * NS

CH = 128
K = 2


def _gather_sc(idx_t, embedding, b, t, d):
    n_total = b * t
    n_units = n_total // CH
    n_su = n_units // K
    su_per_w = n_su // NW
    bh_n = b // CH
    dh_n = d // 8
    tbl = d * CH * K

    mesh = plsc.VectorSubcoreMesh(core_axis_name="c", subcore_axis_name="s")

    @functools.partial(
        pl.kernel,
        mesh=mesh,
        out_type=jax.ShapeDtypeStruct((n_total * d,), jnp.float32),
        compiler_params=pltpu.CompilerParams(
            use_tc_tiling_on_sc=False, needs_layout_passes=False),
        scratch_types=[
            pltpu.VMEM((su_per_w * K * CH,), jnp.int32),
            pltpu.VMEM((K * CH, d), jnp.float32),
            pltpu.VMEM((K * CH, d), jnp.float32),
            pltpu.VMEM((tbl,), jnp.float32),
            pltpu.VMEM((tbl,), jnp.float32),
            pltpu.SemaphoreType.DMA,
            pltpu.SemaphoreType.DMA,
            pltpu.SemaphoreType.DMA,
            pltpu.SemaphoreType.DMA,
        ],
    )
    def k(idx_hbm, table_hbm, out_hbm, idx_v, rows0, rows1, tb0, tb1,
          g0, g1, w0, w1):
        wid = lax.axis_index("s") * NC + lax.axis_index("c")
        sbase = wid * su_per_w
        pltpu.sync_copy(
            idx_hbm.at[pl.ds(sbase * K * CH, su_per_w * K * CH)], idx_v)

        iota = lax.iota(jnp.int32, 16)
        skews = [lax.rem(iota + dd, 16) for dd in range(16)]
        svs = [[((iota + c0) // 8) * (K * CH * 8) + ((iota + c0) % 8) * CH
                + skews[dd] for dd in range(16)]
               for c0 in range(0, d, 16)]

        def gather(ls, buf, sem):
            return pltpu.make_async_copy(
                table_hbm.at[idx_v.at[pl.ds(ls * (K * CH), K * CH)]], buf, sem)

        def transpose(rows, tb):
            for kk in range(K):
                def tblk(ii, _, kk=kk):
                    bi0 = ii * 16
                    for j, c0 in enumerate(range(0, d, 16)):
                        for dd in range(16):
                            rv = skews[dd] + (kk * CH + bi0)
                            v = plsc.load_gather(rows, [rv, iota + c0])
                            plsc.store_scatter(
                                tb, [svs[j][dd] + (kk * CH * 8 + bi0)], v)
                    return 0
                lax.fori_loop(0, CH // 16, tblk, 0)

        def writes(ls, tb, sem):
            u = (sbase + ls) * K
            row = (u // bh_n) * (dh_n * bh_n) + lax.rem(u, bh_n)
            return [pltpu.make_async_copy(
                        tb.at[pl.ds(ch * (K * CH * 8), K * CH * 8)],
                        out_hbm.at[pl.ds((row + ch * bh_n) * (8 * CH),
                                         K * CH * 8)],
                        sem)
                    for ch in range(dh_n)]

        gather(0, rows0, g0).start()
        gather(1, rows1, g1).start()

        def su(i, ls, rows, tb, gsem, wsem):
            gather(ls, rows, gsem).wait()
            transpose(rows, tb)
            gather(lax.rem(ls + 2, su_per_w), rows, gsem).start()
            for cp in writes(ls, tb, wsem):
                cp.start()

        def body(i, _):
            ls = 2 * i
            su(i, ls, rows0, tb0, g0, w0)
            su(i, ls + 1, rows1, tb1, g1, w1)
            for cp in writes(ls, tb0, w0) + writes(ls + 1, tb1, w1):
                cp.wait()
            return 0

        lax.fori_loop(0, su_per_w // 2, body, 0)

        gather(0, rows0, g0).wait()
        gather(1, rows1, g1).wait()

    return k(idx_t, embedding)


def kernel(state, embedding):
    b, t = state.shape
    _, d = embedding.shape
    idx_t = state.T.reshape(b * t)
    out1 = _gather_sc(idx_t, embedding, b, t, d)
    out5 = out1.reshape(t, d // 8, b // CH, 8, CH)
    return out5.transpose(2, 4, 0, 1, 3).reshape(b, t, d)
